# baseline (speedup 1.0000x reference)
import jax
import jax.numpy as jnp
from jax import lax
from jax.experimental import pallas as pl
from jax.experimental.pallas import tpu as pltpu
from jax.experimental.pallas import tpu_sc as plsc

B = 4096
L = 200
D = 32
TOK = 1000000
NC = 2
NS = 16
NW = NC * NS
LANES = 16

NBLK = TOK // 128 + 1
SLOTS = (NBLK + NW - 1) // NW
TAIL_T0 = (TOK // 128) * 128


def _tr_body(tt_hbm, tail_hbm, lin_hbm, in_v0, in_v1, out_v0, out_v1,
             semi0, semi1, semo0, semo1):
    w = lax.axis_index("s") * NC + lax.axis_index("c")
    col32 = lax.iota(jnp.int32, LANES) * D

    def t0_of(slot):
        return pl.multiple_of((slot * NW + w) * 128, 128)

    def fire_in(slot, in_v, semi):
        t0 = t0_of(slot)

        @pl.when(t0 + 128 <= TOK)
        def _():
            pltpu.async_copy(tt_hbm.at[:, pl.ds(t0, 128)], in_v, semi)

        @pl.when(t0 == TAIL_T0)
        def _():
            pltpu.async_copy(tail_hbm, in_v, semi)

    def process(slot, in_v, out_v, semi, semo):
        t0 = t0_of(slot)
        is_tail = t0 == TAIL_T0

        @pl.when(jnp.logical_or(t0 + 128 <= TOK, is_tail))
        def _():
            pltpu.make_async_copy(tt_hbm.at[:, pl.ds(0, 128)], in_v,
                                  semi).wait()

            @pl.when(slot >= 2)
            def _():
                pltpu.make_async_copy(lin_hbm.at[pl.ds(0, 128 * D)],
                                      out_v, semo).wait()

            for d in range(D):
                for g in range(8):
                    vals = in_v[d, pl.ds(g * LANES, LANES)]
                    plsc.store_scatter(out_v,
                                       [col32 + (g * LANES * D + d)], vals)

            @pl.when(t0 + 128 <= TOK)
            def _():
                pltpu.async_copy(out_v, lin_hbm.at[pl.ds(t0 * D, 128 * D)],
                                 semo)

            @pl.when(is_tail)
            def _():
                pltpu.async_copy(out_v.at[pl.ds(64 * D, 64 * D)],
                                 lin_hbm.at[pl.ds(t0 * D, 64 * D)], semo)

    fire_in(0, in_v0, semi0)

    @pl.loop(0, (SLOTS + 1) // 2)
    def _(k):
        s = k * 2
        fire_in(s + 1, in_v1, semi1)
        process(s, in_v0, out_v0, semi0, semo0)
        fire_in(s + 2, in_v0, semi0)
        process(s + 1, in_v1, out_v1, semi1, semo1)

    for last, out_v, semo in ((SLOTS - 2, out_v1, semo1),
                              (SLOTS - 1, out_v0, semo0)):
        t0 = t0_of(last)

        @pl.when(t0 + 128 <= TOK)
        def _():
            pltpu.make_async_copy(lin_hbm.at[pl.ds(0, 128 * D)],
                                  out_v, semo).wait()

        @pl.when(t0 == TAIL_T0)
        def _():
            pltpu.make_async_copy(lin_hbm.at[pl.ds(0, 64 * D)],
                                  out_v.at[pl.ds(0, 64 * D)], semo).wait()


BPW = B // NW
UL = 4
LC = L // (2 * UL)
OUTN = B * L * D


def _gat_body(xt_hbm, lin_hbm, pos_hbm, out_hbm, idx_v, rows_v, obuf_v,
              pos_v, semg, semo):
    w = lax.axis_index("s") * NC + lax.axis_index("c")
    b0 = w * BPW
    iota128 = lax.iota(jnp.int32, LANES) * BPW

    pltpu.sync_copy(pos_hbm, pos_v)

    @pl.loop(0, LC)
    def _(lc):
        l0 = lc * 2 * UL
        pltpu.sync_copy(xt_hbm.at[pl.ds(l0, 2 * UL), pl.ds(b0, BPW)], idx_v)

        for half in range(2):
            @pl.when(lc > 0)
            def _():
                pltpu.make_async_copy(out_hbm.at[pl.ds(0, UL * D * BPW)],
                                      obuf_v.at[half], semo.at[half]).wait()

            for li in range(UL):
                pltpu.async_copy(lin_hbm.at[idx_v.at[half * UL + li]],
                                 rows_v.at[half, li], semg.at[half])

        for half in range(2):
            for li in range(UL):
                pltpu.make_async_copy(lin_hbm.at[pl.ds(0, BPW)],
                                      rows_v.at[half, li],
                                      semg.at[half]).wait()

            for li in range(UL):
                lpos = l0 + half * UL + li
                p0 = pos_v[pl.ds(lpos * D, LANES)]
                p1 = pos_v[pl.ds(lpos * D + LANES, LANES)]

                @pl.loop(0, BPW // 8)
                def _(bg):
                    bi0 = bg * 8
                    for bo in range(8):
                        bi = bi0 + bo
                        v0 = rows_v[half, li, bi, pl.ds(0, LANES)] * p0
                        v1 = rows_v[half, li, bi, pl.ds(LANES, LANES)] * p1
                        base = li * (D * BPW) + bi
                        plsc.store_scatter(obuf_v.at[half],
                                           [iota128 + base], v0)
                        plsc.store_scatter(obuf_v.at[half],
                                           [iota128 + (base + LANES * BPW)],
                                           v1)

            for li in range(UL):
                lpos = l0 + half * UL + li
                for di in range(4):
                    off = ((lpos * 4 + di) * 32 + w) * 1024
                    pltpu.async_copy(
                        obuf_v.at[half, pl.ds((li * 4 + di) * 1024, 1024)],
                        out_hbm.at[pl.ds(off, 1024)], semo.at[half])

    @pl.loop(0, 2)
    def _(half):
        pltpu.make_async_copy(out_hbm.at[pl.ds(0, UL * D * BPW)],
                              obuf_v.at[half], semo.at[half]).wait()


_MESH = plsc.VectorSubcoreMesh(core_axis_name="c", subcore_axis_name="s")


@jax.jit
def _encode(x, token_table, pos_table):
    tt_t = jnp.transpose(token_table)
    tt_tail = lax.slice(tt_t, (0, TOK - 128), (D, TOK))
    xt = jnp.transpose(x)
    pos_lin = pos_table.reshape(L * D)

    tr = pl.kernel(
        _tr_body,
        out_type=jax.ShapeDtypeStruct((TOK * D,), jnp.float32),
        mesh=_MESH,
        compiler_params=pltpu.CompilerParams(use_tc_tiling_on_sc=True,
                                             needs_layout_passes=False),
        scratch_types=[
            pltpu.VMEM((D, 128), jnp.float32),
            pltpu.VMEM((D, 128), jnp.float32),
            pltpu.VMEM((128 * D,), jnp.float32),
            pltpu.VMEM((128 * D,), jnp.float32),
            pltpu.SemaphoreType.DMA,
            pltpu.SemaphoreType.DMA,
            pltpu.SemaphoreType.DMA,
            pltpu.SemaphoreType.DMA,
        ],
    )
    lin = tr(tt_t, tt_tail)

    gat = pl.kernel(
        _gat_body,
        out_type=jax.ShapeDtypeStruct((OUTN,), jnp.float32),
        mesh=_MESH,
        compiler_params=pltpu.CompilerParams(use_tc_tiling_on_sc=False,
                                             needs_layout_passes=False),
        scratch_types=[
            pltpu.VMEM((2 * UL, BPW), jnp.int32),
            pltpu.VMEM((2, UL, BPW, D), jnp.float32),
            pltpu.VMEM((2, UL * D * BPW), jnp.float32),
            pltpu.VMEM((L * D,), jnp.float32),
            pltpu.SemaphoreType.DMA((2,)),
            pltpu.SemaphoreType.DMA((2,)),
        ],
    )
    out_flat = gat(xt, lin.reshape(TOK, D), pos_lin)

    out5 = out_flat.reshape(L, 4, 32, 8, BPW)
    return jnp.transpose(out5, (2, 4, 0, 1, 3)).reshape(B, L, D)


def kernel(x, token_table, pos_table):
    return _encode(x.astype(jnp.int32), token_table, pos_table)


# speedup vs baseline: 1.0406x; 1.0406x over previous
import jax
import jax.numpy as jnp
from jax import lax
from jax.experimental import pallas as pl
from jax.experimental.pallas import tpu as pltpu
from jax.experimental.pallas import tpu_sc as plsc

B = 4096
L = 200
D = 32
TOK = 1000000
NC = 2
NS = 16
NW = NC * NS
LANES = 16

BLK = 512
NFULL = TOK // BLK
TAIL_ID = NFULL
NSLOT = 62
TAIL_T0 = NFULL * BLK


def _tr_body(tt_hbm, tail_hbm, lin_hbm, in_v0, in_v1, out_v0, out_v1,
             semi0, semi1, semo0, semo1):
    w = lax.axis_index("s") * NC + lax.axis_index("c")
    col32 = lax.iota(jnp.int32, LANES) * D

    def bid_of(n):
        return n * NW + w

    def fire_in(n, in_v, semi):
        bid = bid_of(n)
        t0 = pl.multiple_of(bid * BLK, 128)

        @pl.when(bid < NFULL)
        def _():
            pltpu.async_copy(tt_hbm.at[:, pl.ds(t0, BLK)], in_v, semi)

        @pl.when(bid == TAIL_ID)
        def _():
            pltpu.async_copy(tail_hbm, in_v.at[:, pl.ds(0, 128)], semi)

    def process(n, in_v, out_v, semi, semo):
        bid = bid_of(n)
        t0 = bid * BLK

        @pl.when(bid < NFULL)
        def _():
            pltpu.make_async_copy(tt_hbm.at[:, pl.ds(0, BLK)], in_v,
                                  semi).wait()

            @pl.when(n >= 2)
            def _():
                pltpu.make_async_copy(lin_hbm.at[pl.ds(0, BLK * D)],
                                      out_v, semo).wait()

            @pl.loop(0, D)
            def _(d):
                for g in range(BLK // LANES):
                    vals = in_v[d, pl.ds(g * LANES, LANES)]
                    plsc.store_scatter(out_v,
                                       [col32 + (g * LANES * D + d)], vals)

            pltpu.async_copy(out_v, lin_hbm.at[pl.ds(t0 * D, BLK * D)], semo)

        @pl.when(bid == TAIL_ID)
        def _():
            pltpu.make_async_copy(tail_hbm, in_v.at[:, pl.ds(0, 128)],
                                  semi).wait()

            @pl.when(n >= 2)
            def _():
                pltpu.make_async_copy(lin_hbm.at[pl.ds(0, BLK * D)],
                                      out_v, semo).wait()

            @pl.loop(0, D)
            def _(d):
                for g in range(8):
                    vals = in_v[d, pl.ds(g * LANES, LANES)]
                    plsc.store_scatter(out_v,
                                       [col32 + (g * LANES * D + d)], vals)

            pltpu.async_copy(out_v.at[pl.ds(64 * D, 64 * D)],
                             lin_hbm.at[pl.ds(TAIL_T0 * D, 64 * D)], semo)

    fire_in(0, in_v0, semi0)

    @pl.loop(0, NSLOT // 2)
    def _(k):
        n = k * 2
        fire_in(n + 1, in_v1, semi1)
        process(n, in_v0, out_v0, semi0, semo0)
        fire_in(n + 2, in_v0, semi0)
        process(n + 1, in_v1, out_v1, semi1, semo1)

    for last, out_v, semo in ((NSLOT - 2, out_v0, semo0),
                              (NSLOT - 1, out_v1, semo1)):
        bid = bid_of(last)

        @pl.when(bid < NFULL)
        def _():
            pltpu.make_async_copy(lin_hbm.at[pl.ds(0, BLK * D)],
                                  out_v, semo).wait()

        @pl.when(bid == TAIL_ID)
        def _():
            pltpu.make_async_copy(lin_hbm.at[pl.ds(0, 64 * D)],
                                  out_v.at[pl.ds(64 * D, 64 * D)],
                                  semo).wait()


BPW = B // NW
GL = 8
NG = L // GL
OUTN = B * L * D


def _gat_body(xt_hbm, lin_hbm, pos_hbm, out_hbm, idx_v0, idx_v1, rows_v0,
              rows_v1, obuf_v0, obuf_v1, pos_v, semi0, semi1, semg0, semg1,
              semo0, semo1):
    w = lax.axis_index("s") * NC + lax.axis_index("c")
    b0 = w * BPW
    iota128 = lax.iota(jnp.int32, LANES) * BPW
    idx_bufs = (idx_v0, idx_v1)
    rows_bufs = (rows_v0, rows_v1)
    obuf_bufs = (obuf_v0, obuf_v1)
    semi = (semi0, semi1)
    semg = (semg0, semg1)
    semo = (semo0, semo1)

    pltpu.sync_copy(pos_hbm, pos_v)

    def fire_idx(g, p):
        @pl.when(g < NG)
        def _():
            pltpu.async_copy(
                xt_hbm.at[pl.ds(pl.multiple_of(g * GL, GL), GL),
                          pl.ds(b0, BPW)], idx_bufs[p], semi[p])

    def wait_idx(p):
        pltpu.make_async_copy(xt_hbm.at[pl.ds(0, GL), pl.ds(0, BPW)],
                              idx_bufs[p], semi[p]).wait()

    def fire_gathers(p):
        for li in range(GL):
            pltpu.async_copy(lin_hbm.at[idx_bufs[p].at[li]],
                             rows_bufs[p].at[li], semg[p])

    def wait_gathers(p):
        for li in range(GL):
            pltpu.make_async_copy(lin_hbm.at[pl.ds(0, BPW)],
                                  rows_bufs[p].at[li], semg[p]).wait()

    def compute(g, p):
        l0 = g * GL
        rows_v = rows_bufs[p]
        for half in range(2):
            obuf = obuf_bufs[half]

            @pl.when(g > 0)
            def _():
                pltpu.make_async_copy(out_hbm.at[pl.ds(0, 4 * D * BPW)],
                                      obuf, semo[half]).wait()

            for li in range(4):
                lpos = l0 + half * 4 + li
                p0 = pos_v[pl.ds(lpos * D, LANES)]
                p1 = pos_v[pl.ds(lpos * D + LANES, LANES)]

                @pl.loop(0, BPW // 8)
                def _(bg):
                    bi0 = bg * 8
                    for bo in range(8):
                        bi = bi0 + bo
                        r = half * 4 + li
                        v0 = rows_v[r, bi, pl.ds(0, LANES)] * p0
                        v1 = rows_v[r, bi, pl.ds(LANES, LANES)] * p1
                        base = li * (D * BPW) + bi
                        plsc.store_scatter(obuf, [iota128 + base], v0)
                        plsc.store_scatter(obuf,
                                           [iota128 + (base + LANES * BPW)],
                                           v1)

            for li in range(4):
                lpos = l0 + half * 4 + li
                for di in range(4):
                    off = ((lpos * 4 + di) * D + w) * 1024
                    pltpu.async_copy(obuf.at[pl.ds((li * 4 + di) * 1024,
                                                   1024)],
                                     out_hbm.at[pl.ds(off, 1024)], semo[half])

    def step(g, cur, nxt):
        @pl.when(g + 1 < NG)
        def _():
            wait_idx(nxt)
            fire_gathers(nxt)

        wait_gathers(cur)
        fire_idx(g + 2, cur)
        compute(g, cur)

    pltpu.sync_copy(xt_hbm.at[pl.ds(0, GL), pl.ds(b0, BPW)], idx_bufs[0])
    fire_gathers(0)
    fire_idx(1, 1)

    @pl.loop(0, NG // 2)
    def _(k):
        g = k * 2
        step(g, 0, 1)
        step(g + 1, 1, 0)

    step(NG - 1, 0, 1)

    for half in range(2):
        pltpu.make_async_copy(out_hbm.at[pl.ds(0, 4 * D * BPW)],
                              obuf_bufs[half], semo[half]).wait()


_MESH = plsc.VectorSubcoreMesh(core_axis_name="c", subcore_axis_name="s")


@jax.jit
def _encode(x, token_table, pos_table):
    tt_t = jnp.transpose(token_table)
    tt_tail = lax.slice(tt_t, (0, TOK - 128), (D, TOK))
    xt = jnp.transpose(x)
    pos_lin = pos_table.reshape(L * D)

    tr = pl.kernel(
        _tr_body,
        out_type=jax.ShapeDtypeStruct((TOK * D,), jnp.float32),
        mesh=_MESH,
        compiler_params=pltpu.CompilerParams(use_tc_tiling_on_sc=True,
                                             needs_layout_passes=False),
        scratch_types=[
            pltpu.VMEM((D, BLK), jnp.float32),
            pltpu.VMEM((D, BLK), jnp.float32),
            pltpu.VMEM((BLK * D,), jnp.float32),
            pltpu.VMEM((BLK * D,), jnp.float32),
            pltpu.SemaphoreType.DMA,
            pltpu.SemaphoreType.DMA,
            pltpu.SemaphoreType.DMA,
            pltpu.SemaphoreType.DMA,
        ],
    )
    lin = tr(tt_t, tt_tail)

    gat = pl.kernel(
        _gat_body,
        out_type=jax.ShapeDtypeStruct((OUTN,), jnp.float32),
        mesh=_MESH,
        compiler_params=pltpu.CompilerParams(use_tc_tiling_on_sc=False,
                                             needs_layout_passes=False),
        scratch_types=[
            pltpu.VMEM((GL, BPW), jnp.int32),
            pltpu.VMEM((GL, BPW), jnp.int32),
            pltpu.VMEM((GL, BPW, D), jnp.float32),
            pltpu.VMEM((GL, BPW, D), jnp.float32),
            pltpu.VMEM((4 * D * BPW,), jnp.float32),
            pltpu.VMEM((4 * D * BPW,), jnp.float32),
            pltpu.VMEM((L * D,), jnp.float32),
            pltpu.SemaphoreType.DMA,
            pltpu.SemaphoreType.DMA,
            pltpu.SemaphoreType.DMA,
            pltpu.SemaphoreType.DMA,
            pltpu.SemaphoreType.DMA,
            pltpu.SemaphoreType.DMA,
        ],
    )
    out_flat = gat(xt, lin.reshape(TOK, D), pos_lin)

    out5 = out_flat.reshape(L, 4, D, 8, BPW)
    return jnp.transpose(out5, (2, 4, 0, 1, 3)).reshape(B, L, D)


def kernel(x, token_table, pos_table):
    return _encode(x.astype(jnp.int32), token_table, pos_table)


# speedup vs baseline: 1.1727x; 1.1269x over previous
import jax
import jax.numpy as jnp
from jax import lax
from jax.experimental import pallas as pl
from jax.experimental.pallas import tpu as pltpu
from jax.experimental.pallas import tpu_sc as plsc

B = 4096
L = 200
D = 32
NC = 2
NS = 16
NW = NC * NS
BPW = B // NW
RB = 8
NCHUNKS = BPW // RB
SUB = 40
KSUB = L // SUB
LANES = 16


def _body(x_hbm, tok_hbm, pos_hbm, out_hbm, idx_v, rows_v, pos_v, sem_g):
    wid = lax.axis_index("s") * NC + lax.axis_index("c")
    pltpu.sync_copy(pos_hbm, pos_v)

    @pl.loop(0, NCHUNKS)
    def _chunk(c):
        b0 = wid * BPW + c * RB

        pltpu.sync_copy(x_hbm.at[pl.ds(b0, RB)], idx_v)

        for r in range(RB):
            for j in range(KSUB):
                pltpu.async_copy(
                    tok_hbm.at[idx_v.at[r, pl.ds(j * SUB, SUB)]],
                    rows_v.at[r, pl.ds(j * SUB, SUB)],
                    sem_g,
                )

        pltpu.make_async_copy(out_hbm.at[pl.ds(0, RB)], rows_v, sem_g).wait()

        @pl.loop(0, L)
        def _mul(l):
            p0 = pos_v[l, pl.ds(0, LANES)]
            p1 = pos_v[l, pl.ds(LANES, LANES)]
            for r in range(RB):
                rows_v[r, l, pl.ds(0, LANES)] = rows_v[r, l, pl.ds(0, LANES)] * p0
                rows_v[r, l, pl.ds(LANES, LANES)] = (
                    rows_v[r, l, pl.ds(LANES, LANES)] * p1
                )

        pltpu.sync_copy(rows_v, out_hbm.at[pl.ds(b0, RB)])


@jax.jit
def _encode(x, token_table, pos_table):
    mesh = plsc.VectorSubcoreMesh(core_axis_name="c", subcore_axis_name="s")
    k = pl.kernel(
        _body,
        out_type=jax.ShapeDtypeStruct((B, L, D), jnp.float32),
        mesh=mesh,
        compiler_params=pltpu.CompilerParams(use_tc_tiling_on_sc=False),
        scratch_types=[
            pltpu.VMEM((RB, L), jnp.int32),
            pltpu.VMEM((RB, L, D), jnp.float32),
            pltpu.VMEM((L, D), jnp.float32),
            pltpu.SemaphoreType.DMA,
        ],
    )
    return k(x, token_table, pos_table)


def kernel(x, token_table, pos_table):
    return _encode(x.astype(jnp.int32), token_table, pos_table)
